# baseline (speedup 1.0000x reference)
import jax
import jax.numpy as jnp
from jax import lax
from jax.experimental import pallas as pl
from jax.experimental.pallas import tpu as pltpu
from jax.experimental.pallas import tpu_sc as plsc

_MOM = 0.2
_B = 1024
_V = 8192
_D = 64 * 16 * 8
_NC = 2
_NS = 16
_NW = _NC * _NS
_BPW = _B // _NW
_CHUNK = 2
_NCHUNK = _BPW // _CHUNK
_L = 16
_SLICES = _D // _L


def _lane_sum(x):
    lanes = lax.iota(jnp.int32, _L)
    for k in (8, 4, 2, 1):
        perm = lax.bitwise_xor(lanes, jnp.int32(k))
        shuf = lax.gather(
            x, perm[:, None],
            dimension_numbers=lax.GatherDimensionNumbers(
                offset_dims=(), collapsed_slice_dims=(0,),
                start_index_map=(0,)),
            slice_sizes=(1,),
            mode=lax.GatherScatterMode.PROMISE_IN_BOUNDS)
        x = x + shuf
    return x


def _rsqrt_vec(s):
    i = lax.bitcast_convert_type(s, jnp.int32)
    i = jnp.full((_L,), 0x5F3759DF, jnp.int32) - lax.shift_right_logical(i, 1)
    y = lax.bitcast_convert_type(i, jnp.float32)
    for _ in range(4):
        y = y * (1.5 - 0.5 * s * y * y)
    return y


def _process_row(row_ref, x_ref, r):
    zero = jnp.zeros((_L,), jnp.float32)

    @plsc.parallel_loop(0, _SLICES, 4, unroll=4, carry=(zero, zero, zero, zero))
    def accs(i, carry):
        out = []
        for k in range(4):
            sl = pl.ds((i + k) * _L, _L)
            u = _MOM * row_ref[r, sl] + (1.0 - _MOM) * x_ref[r, sl]
            row_ref[r, sl] = u
            out.append(carry[k] + u * u)
        return tuple(out)

    inv = _rsqrt_vec(_lane_sum(accs[0] + accs[1] + accs[2] + accs[3]))

    @plsc.parallel_loop(0, _SLICES, 8, unroll=4)
    def _(i):
        for k in range(8):
            sl = pl.ds((i + k) * _L, _L)
            x_ref[r, sl] = row_ref[r, sl] * inv


def _sc_momentum_update(inputs2d, idx3, cam2d):
    mesh = plsc.VectorSubcoreMesh(core_axis_name="c", subcore_axis_name="s")

    def body(in_hbm, idx_hbm, cam_hbm, out_hbm, flag_hbm,
             idx_v, row0, row1, row2, x0, x1, x2, z_v,
             gsem0, gsem1, gsem2, xsem0, xsem1, xsem2, ssem0, ssem1, ssem2):
        rows = (row0, row1, row2)
        xs = (x0, x1, x2)
        gsems = (gsem0, gsem1, gsem2)
        xsems = (xsem0, xsem1, xsem2)
        ssems = (ssem0, ssem1, ssem2)
        wid = lax.axis_index("s") * _NC + lax.axis_index("c")
        pltpu.sync_copy(idx_hbm.at[wid], idx_v)

        def issue(c):
            b = c % 3
            hg = pltpu.async_copy(cam_hbm.at[idx_v.at[c]], rows[b], gsems[b])
            hx = pltpu.async_copy(
                in_hbm.at[pl.ds(wid * _BPW + c * _CHUNK, _CHUNK)], xs[b], xsems[b])
            return hg, hx

        h_in = [None] * _NCHUNK
        h_sc = [None] * _NCHUNK
        h_in[0] = issue(0)
        h_in[1] = issue(1)
        for c in range(_NCHUNK):
            b = c % 3
            if c + 2 < _NCHUNK:
                if c >= 1:
                    h_sc[c - 1].wait()
                h_in[c + 2] = issue(c + 2)
            h_in[c][0].wait()
            h_in[c][1].wait()
            for r in range(_CHUNK):
                _process_row(rows[b], xs[b], r)
            h_sc[c] = pltpu.async_copy(xs[b], out_hbm.at[idx_v.at[c]], ssems[b])
        h_sc[_NCHUNK - 2].wait()
        h_sc[_NCHUNK - 1].wait()

        @pl.when(wid == 0)
        def _():
            z_v[...] = jnp.zeros((_L,), jnp.int32)
            pltpu.sync_copy(z_v, flag_hbm)

    f = pl.kernel(
        body,
        out_type=(
            jax.ShapeDtypeStruct((_V, _D), jnp.float32),
            jax.ShapeDtypeStruct((_L,), jnp.int32),
        ),
        mesh=mesh,
        scratch_types=[
            pltpu.VMEM((_NCHUNK, _CHUNK), jnp.int32),
            pltpu.VMEM((_CHUNK, _D), jnp.float32),
            pltpu.VMEM((_CHUNK, _D), jnp.float32),
            pltpu.VMEM((_CHUNK, _D), jnp.float32),
            pltpu.VMEM((_CHUNK, _D), jnp.float32),
            pltpu.VMEM((_CHUNK, _D), jnp.float32),
            pltpu.VMEM((_CHUNK, _D), jnp.float32),
            pltpu.VMEM((_L,), jnp.int32),
            pltpu.SemaphoreType.DMA,
            pltpu.SemaphoreType.DMA,
            pltpu.SemaphoreType.DMA,
            pltpu.SemaphoreType.DMA,
            pltpu.SemaphoreType.DMA,
            pltpu.SemaphoreType.DMA,
            pltpu.SemaphoreType.DMA,
            pltpu.SemaphoreType.DMA,
            pltpu.SemaphoreType.DMA,
        ],
    )
    return f(inputs2d, idx3, cam2d)


def kernel(inputs, indexes, cam_features, labels):
    del labels
    inputs2d = inputs.reshape(_B, _D)
    idx3 = indexes.reshape(_NW, _NCHUNK, _CHUNK)
    cam2d = cam_features.reshape(_V, _D)
    _, flag = _sc_momentum_update(inputs2d, idx3, cam2d)
    return flag[0]


# speedup vs baseline: 1.0122x; 1.0122x over previous
import jax
import jax.numpy as jnp
from jax import lax
from jax.experimental import pallas as pl
from jax.experimental.pallas import tpu as pltpu
from jax.experimental.pallas import tpu_sc as plsc

_MOM = 0.2
_B = 1024
_V = 8192
_D = 64 * 16 * 8
_NC = 2
_NS = 16
_NW = _NC * _NS
_BPW = _B // _NW
_CHUNK = 2
_NCHUNK = _BPW // _CHUNK
_L = 16
_SLICES = _D // _L


def _lane_sum(x):
    lanes = lax.iota(jnp.int32, _L)
    for k in (8, 4, 2, 1):
        perm = lax.bitwise_xor(lanes, jnp.int32(k))
        shuf = lax.gather(
            x, perm[:, None],
            dimension_numbers=lax.GatherDimensionNumbers(
                offset_dims=(), collapsed_slice_dims=(0,),
                start_index_map=(0,)),
            slice_sizes=(1,),
            mode=lax.GatherScatterMode.PROMISE_IN_BOUNDS)
        x = x + shuf
    return x


def _rsqrt_vec(s):
    i = lax.bitcast_convert_type(s, jnp.int32)
    i = jnp.full((_L,), 0x5F3759DF, jnp.int32) - lax.shift_right_logical(i, 1)
    y = lax.bitcast_convert_type(i, jnp.float32)
    for _ in range(4):
        y = y * (1.5 - 0.5 * s * y * y)
    return y


def _process_row(row_ref, x_ref, r):
    zero = jnp.zeros((_L,), jnp.float32)

    @plsc.parallel_loop(0, _SLICES, 4, unroll=4, carry=(zero, zero, zero, zero))
    def accs(i, carry):
        out = []
        for k in range(4):
            sl = pl.ds((i + k) * _L, _L)
            u = _MOM * row_ref[r, sl] + (1.0 - _MOM) * x_ref[r, sl]
            row_ref[r, sl] = u
            out.append(carry[k] + u * u)
        return tuple(out)

    inv = _rsqrt_vec(_lane_sum(accs[0] + accs[1] + accs[2] + accs[3]))

    @plsc.parallel_loop(0, _SLICES, 8, unroll=4)
    def _(i):
        for k in range(8):
            sl = pl.ds((i + k) * _L, _L)
            row_ref[r, sl] = row_ref[r, sl] * inv


def _sc_momentum_update(inputs2d, idx3, cam2d):
    mesh = plsc.VectorSubcoreMesh(core_axis_name="c", subcore_axis_name="s")

    def body(in_hbm, idx_hbm, cam_hbm, out_hbm, flag_hbm,
             idx_v, row0, row1, row2, x0, x1, x2, z_v,
             gsem0, gsem1, gsem2, xsem0, xsem1, xsem2, ssem0, ssem1, ssem2):
        rows = (row0, row1, row2)
        xs = (x0, x1, x2)
        gsems = (gsem0, gsem1, gsem2)
        xsems = (xsem0, xsem1, xsem2)
        ssems = (ssem0, ssem1, ssem2)
        wid = lax.axis_index("s") * _NC + lax.axis_index("c")
        pltpu.sync_copy(idx_hbm.at[wid], idx_v)

        def issue(c):
            b = c % 3
            hg = pltpu.async_copy(cam_hbm.at[idx_v.at[c]], rows[b], gsems[b])
            hx = pltpu.async_copy(
                in_hbm.at[pl.ds(wid * _BPW + c * _CHUNK, _CHUNK)], xs[b], xsems[b])
            return hg, hx

        h_in = [None] * _NCHUNK
        h_sc = [None] * _NCHUNK
        h_in[0] = issue(0)
        h_in[1] = issue(1)
        for c in range(_NCHUNK):
            b = c % 3
            if c + 2 < _NCHUNK:
                if c >= 1:
                    h_sc[c - 1].wait()
                h_in[c + 2] = issue(c + 2)
            h_in[c][0].wait()
            h_in[c][1].wait()
            for r in range(_CHUNK):
                _process_row(rows[b], xs[b], r)
            h_sc[c] = pltpu.async_copy(rows[b], out_hbm.at[idx_v.at[c]], ssems[b])
        h_sc[_NCHUNK - 2].wait()
        h_sc[_NCHUNK - 1].wait()

        @pl.when(wid == 0)
        def _():
            z_v[...] = jnp.zeros((_L,), jnp.int32)
            pltpu.sync_copy(z_v, flag_hbm)

    f = pl.kernel(
        body,
        out_type=(
            jax.ShapeDtypeStruct((_V, _D), jnp.float32),
            jax.ShapeDtypeStruct((_L,), jnp.int32),
        ),
        mesh=mesh,
        scratch_types=[
            pltpu.VMEM((_NCHUNK, _CHUNK), jnp.int32),
            pltpu.VMEM((_CHUNK, _D), jnp.float32),
            pltpu.VMEM((_CHUNK, _D), jnp.float32),
            pltpu.VMEM((_CHUNK, _D), jnp.float32),
            pltpu.VMEM((_CHUNK, _D), jnp.float32),
            pltpu.VMEM((_CHUNK, _D), jnp.float32),
            pltpu.VMEM((_CHUNK, _D), jnp.float32),
            pltpu.VMEM((_L,), jnp.int32),
            pltpu.SemaphoreType.DMA,
            pltpu.SemaphoreType.DMA,
            pltpu.SemaphoreType.DMA,
            pltpu.SemaphoreType.DMA,
            pltpu.SemaphoreType.DMA,
            pltpu.SemaphoreType.DMA,
            pltpu.SemaphoreType.DMA,
            pltpu.SemaphoreType.DMA,
            pltpu.SemaphoreType.DMA,
        ],
    )
    return f(inputs2d, idx3, cam2d)


def kernel(inputs, indexes, cam_features, labels):
    del labels
    inputs2d = inputs.reshape(_B, _D)
    idx3 = indexes.reshape(_NW, _NCHUNK, _CHUNK)
    cam2d = cam_features.reshape(_V, _D)
    _, flag = _sc_momentum_update(inputs2d, idx3, cam2d)
    return flag[0]
